# Initial kernel scaffold; baseline (speedup 1.0000x reference)
import jax
import jax.numpy as jnp
from jax.experimental import pallas as pl


def kernel(text_embeddings, mel_embeddings, text_mask, mel_mask, Wq, Wk, Wv, w_dur, Wt, Wm):
    raise NotImplementedError("write your pallas kernel here")


# speedup vs baseline: 1.4564x; 1.4564x over previous
import functools

import jax
import jax.numpy as jnp
from jax.experimental import pallas as pl
from jax.experimental.pallas import tpu as pltpu

_B, _I, _J = 8, 256, 2048
_CT, _CM, _A = 512, 512, 256
_D = 3
_K = (2 * _D + 1) * _I

_NEG = -1e9
_BF = jnp.bfloat16
_F32 = jnp.float32


def _bf(x):
    return x.astype(_BF)


def _dot_bf(a, b):
    return jax.lax.dot_general(
        _bf(a), _bf(b), (((1,), (0,)), ((), ())), preferred_element_type=_F32
    )


def _dot_bf_t(a, b):
    return jax.lax.dot_general(
        _bf(a), _bf(b), (((1,), (1,)), ((), ())), preferred_element_type=_F32
    )


def _split3(x):
    h = _bf(x)
    r = x - h.astype(_F32)
    m = _bf(r)
    l = _bf(r - m.astype(_F32))
    return h, m, l


def _onehot_dot(e_bf, x):
    h, m, l = _split3(x)
    acc = jax.lax.dot_general(e_bf, h, (((1,), (0,)), ((), ())),
                              preferred_element_type=_F32)
    acc += jax.lax.dot_general(e_bf, m, (((1,), (0,)), ((), ())),
                               preferred_element_type=_F32)
    acc += jax.lax.dot_general(e_bf, l, (((1,), (0,)), ((), ())),
                               preferred_element_type=_F32)
    return acc


def _cumsum_lanes(x):
    n = x.shape[-1]
    d = 1
    while d < n:
        shifted = jnp.concatenate(
            [jnp.zeros((1, d), x.dtype), x[:, : n - d]], axis=1)
        x = x + shifted
        d *= 2
    return x


def _cummax_lanes(x):
    n = x.shape[-1]
    d = 1
    while d < n:
        shifted = jnp.concatenate(
            [jnp.full((1, d), -jnp.inf, x.dtype), x[:, : n - d]], axis=1)
        x = jnp.maximum(x, shifted)
        d *= 2
    return x


def _cumsum_sublanes(x):
    n = x.shape[0]
    d = 1
    while d < n:
        shifted = jnp.concatenate(
            [jnp.zeros((d, 1), x.dtype), x[: n - d, :]], axis=0)
        x = x + shifted
        d *= 2
    return x


def _s1_kernel(text_ref, mel_ref, wq_ref, wk_ref, wv_ref, wdur_ref,
               m_ref, cnt_ref):
    text = text_ref[0]
    mel = mel_ref[0]
    q = _dot_bf(text, wq_ref[...])
    k = _dot_bf(mel, wk_ref[...])
    v = _dot_bf(mel, wv_ref[...])
    s = _dot_bf_t(q, k) * (1.0 / 16.0)
    s = s - jnp.max(s, axis=1, keepdims=True)
    es = jnp.exp(s)
    attn = es / jnp.sum(es, axis=1, keepdims=True)
    ctx = _dot_bf(attn, v)
    logits = _dot_bf_t(wdur_ref[...], ctx)
    logits = logits - jnp.max(logits, axis=1, keepdims=True)
    el = jnp.exp(logits)
    p = el / jnp.sum(el, axis=1, keepdims=True)
    cum = _cumsum_lanes(p * float(_J))
    ci = jnp.round(cum)
    lane = jax.lax.broadcasted_iota(jnp.int32, (1, _I), 1)
    ci = jnp.where(lane == _I - 1, float(_J), ci)
    ci = _cummax_lanes(ci)
    b_row = (ci - 1.0).astype(jnp.int32)
    f_col = jax.lax.broadcasted_iota(jnp.int32, (_J, _I), 0)
    hit = jnp.abs(f_col - b_row) <= _D
    m_col = jnp.any(hit, axis=1, keepdims=True).astype(_F32)
    m_ref[0] = m_col
    cnt_ref[0] = _cumsum_sublanes(m_col)


def _s2_kernel(text_ref, mel_ref, wt_ref, wm_ref, m_ref, cnt_ref,
               pf_ref, hard_ref, exp_ref):
    text = text_ref[0]
    mel = mel_ref[0]
    m_row = m_ref[0]
    cnti = cnt_ref[0]
    k_col = jax.lax.broadcasted_iota(jnp.int32, (_K, 1), 0)
    pos_row = (cnti - 1.0).astype(jnp.int32)
    member_row = m_row > 0.0

    chunk = 512
    sel = jnp.zeros((_K, _CM), _F32)
    for c0 in range(0, _J, chunk):
        e_bf = jnp.where(
            member_row[:, c0:c0 + chunk]
            & (pos_row[:, c0:c0 + chunk] == k_col),
            1.0, 0.0).astype(_BF)
        sel = sel + _onehot_dot(e_bf, mel[c0:c0 + chunk, :])

    km = _dot_bf(sel, wm_ref[...])
    qt = _dot_bf(text, wt_ref[...])
    energy = _dot_bf_t(qt, km) * (1.0 / 16.0)

    ktot = jnp.max(cnti, axis=1, keepdims=True)
    k_row = jax.lax.broadcasted_iota(jnp.int32, (1, _K), 1)
    valid_row = (k_row.astype(_F32) < ktot)
    em = jnp.where(valid_row, energy, _NEG)
    vmask = valid_row.astype(_F32)

    ex = jnp.exp(em - jnp.max(em, axis=1, keepdims=True))
    mat_p_d = ex / jnp.sum(ex, axis=1, keepdims=True) * vmask

    row_max = jnp.max(em, axis=1, keepdims=True)
    am = jnp.min(jnp.where(em == row_max, k_row, _K), axis=1, keepdims=True)
    hard_ref[0] = jnp.where((k_row == am) & valid_row, 1.0, 0.0)

    ex2 = jnp.exp(em - jnp.max(em, axis=0, keepdims=True))
    w_exp = ex2 / jnp.sum(ex2, axis=0, keepdims=True) * vmask
    exp_ref[0] = jax.lax.dot_general(
        _bf(w_exp), _bf(text), (((0,), (0,)), ((), ())),
        preferred_element_type=_F32)

    cnt_excl = (cnti - m_row).astype(jnp.int32)
    pd_bf = _bf(mat_p_d)
    for c0 in range(0, _J, 256):
        map_bf = jnp.where(
            cnt_excl[:, c0:c0 + 256] == k_col, 1.0, 0.0).astype(_BF)
        pf_ref[0, :, c0:c0 + 256] = jax.lax.dot_general(
            pd_bf, map_bf, (((1,), (0,)), ((), ())),
            preferred_element_type=_F32)


def _stage1(text, mel, wq, wk, wv, wdur):
    out = pl.pallas_call(
        _s1_kernel,
        grid=(_B,),
        in_specs=[
            pl.BlockSpec((1, _I, _CT), lambda b: (b, 0, 0)),
            pl.BlockSpec((1, _J, _CM), lambda b: (b, 0, 0)),
            pl.BlockSpec((_CT, _A), lambda b: (0, 0)),
            pl.BlockSpec((_CM, _A), lambda b: (0, 0)),
            pl.BlockSpec((_CM, _A), lambda b: (0, 0)),
            pl.BlockSpec((1, _A), lambda b: (0, 0)),
        ],
        out_specs=[
            pl.BlockSpec((1, _J, 1), lambda b: (b, 0, 0)),
            pl.BlockSpec((1, _J, 1), lambda b: (b, 0, 0)),
        ],
        out_shape=[
            jax.ShapeDtypeStruct((_B, _J, 1), _F32),
            jax.ShapeDtypeStruct((_B, _J, 1), _F32),
        ],
    )(text, mel, wq, wk, wv, wdur.reshape(1, _A))
    return out


def _stage2(text, mel, wt, wm, m2d, cnt2d):
    return pl.pallas_call(
        _s2_kernel,
        grid=(_B,),
        in_specs=[
            pl.BlockSpec((1, _I, _CT), lambda b: (b, 0, 0)),
            pl.BlockSpec((1, _J, _CM), lambda b: (b, 0, 0)),
            pl.BlockSpec((_CT, _A), lambda b: (0, 0)),
            pl.BlockSpec((_CM, _A), lambda b: (0, 0)),
            pl.BlockSpec((1, 1, _J), lambda b: (b, 0, 0)),
            pl.BlockSpec((1, 1, _J), lambda b: (b, 0, 0)),
        ],
        out_specs=[
            pl.BlockSpec((1, _I, _J), lambda b: (b, 0, 0)),
            pl.BlockSpec((1, _I, _K), lambda b: (b, 0, 0)),
            pl.BlockSpec((1, _K, _CT), lambda b: (b, 0, 0)),
        ],
        out_shape=[
            jax.ShapeDtypeStruct((_B, _I, _J), _F32),
            jax.ShapeDtypeStruct((_B, _I, _K), _F32),
            jax.ShapeDtypeStruct((_B, _K, _CT), _F32),
        ],
    )(text, mel, wt, wm, m2d, cnt2d)


def kernel(text_embeddings, mel_embeddings, text_mask, mel_mask,
           Wq, Wk, Wv, w_dur, Wt, Wm):
    m3d, cnt3d = _stage1(text_embeddings, mel_embeddings, Wq, Wk, Wv, w_dur)
    m2d = m3d.reshape(_B, 1, _J)
    cnt2d = cnt3d.reshape(_B, 1, _J)
    mat_p_f, hard, expanded_text = _stage2(
        text_embeddings, mel_embeddings, Wt, Wm, m2d, cnt2d)
    return mat_p_f, hard, expanded_text


# speedup vs baseline: 1.9625x; 1.3475x over previous
import functools

import jax
import jax.numpy as jnp
from jax.experimental import pallas as pl
from jax.experimental.pallas import tpu as pltpu
from jax.experimental.pallas import tpu_sc as plsc

_B, _I, _J = 8, 256, 2048
_CT, _CM, _A = 512, 512, 256
_D = 3
_K = (2 * _D + 1) * _I

_NEG = -1e9
_BF = jnp.bfloat16
_F32 = jnp.float32


def _bf(x):
    return x.astype(_BF)


def _dot_bf(a, b):
    return jax.lax.dot_general(
        _bf(a), _bf(b), (((1,), (0,)), ((), ())), preferred_element_type=_F32
    )


def _dot_bf_t(a, b):
    return jax.lax.dot_general(
        _bf(a), _bf(b), (((1,), (1,)), ((), ())), preferred_element_type=_F32
    )


def _split3(x):
    h = _bf(x)
    r = x - h.astype(_F32)
    m = _bf(r)
    l = _bf(r - m.astype(_F32))
    return h, m, l


def _onehot_dot(e_bf, x):
    h, m, l = _split3(x)
    acc = jax.lax.dot_general(e_bf, h, (((1,), (0,)), ((), ())),
                              preferred_element_type=_F32)
    acc += jax.lax.dot_general(e_bf, m, (((1,), (0,)), ((), ())),
                               preferred_element_type=_F32)
    acc += jax.lax.dot_general(e_bf, l, (((1,), (0,)), ((), ())),
                               preferred_element_type=_F32)
    return acc


def _cumsum_lanes(x):
    n = x.shape[-1]
    d = 1
    while d < n:
        shifted = jnp.concatenate(
            [jnp.zeros((1, d), x.dtype), x[:, : n - d]], axis=1)
        x = x + shifted
        d *= 2
    return x


def _cummax_lanes(x):
    n = x.shape[-1]
    d = 1
    while d < n:
        shifted = jnp.concatenate(
            [jnp.full((1, d), -jnp.inf, x.dtype), x[:, : n - d]], axis=1)
        x = jnp.maximum(x, shifted)
        d *= 2
    return x


def _cumsum_sublanes(x):
    n = x.shape[0]
    d = 1
    while d < n:
        shifted = jnp.concatenate(
            [jnp.zeros((d, 1), x.dtype), x[: n - d, :]], axis=0)
        x = x + shifted
        d *= 2
    return x


def _s1_kernel(text_ref, mel_ref, wq_ref, wk_ref, wv_ref, wdur_ref,
               m_ref, cnt_ref, uidx_ref):
    text = text_ref[0]
    mel = mel_ref[0]
    q = _dot_bf(text, wq_ref[...])
    k = _dot_bf(mel, wk_ref[...])
    v = _dot_bf(mel, wv_ref[...])
    s = _dot_bf_t(q, k) * (1.0 / 16.0)
    s = s - jnp.max(s, axis=1, keepdims=True)
    es = jnp.exp(s)
    attn = es / jnp.sum(es, axis=1, keepdims=True)
    ctx = _dot_bf(attn, v)
    logits = _dot_bf_t(wdur_ref[...], ctx)
    logits = logits - jnp.max(logits, axis=1, keepdims=True)
    el = jnp.exp(logits)
    p = el / jnp.sum(el, axis=1, keepdims=True)
    cum = _cumsum_lanes(p * float(_J))
    ci = jnp.round(cum)
    lane = jax.lax.broadcasted_iota(jnp.int32, (1, _I), 1)
    ci = jnp.where(lane == _I - 1, float(_J), ci)
    ci = _cummax_lanes(ci)
    b_row = (ci - 1.0).astype(jnp.int32)
    f_col = jax.lax.broadcasted_iota(jnp.int32, (_J, _I), 0)
    hit = jnp.abs(f_col - b_row) <= _D
    m_col = jnp.any(hit, axis=1, keepdims=True).astype(_F32)
    m_ref[0] = m_col
    cnti_col = _cumsum_sublanes(m_col)
    cnt_ref[0] = cnti_col
    b = pl.program_id(0)
    for k0 in range(0, _K, 448):
        k_row = jax.lax.broadcasted_iota(jnp.int32, (1, 448), 1) + k0
        u = jnp.sum((cnti_col <= k_row.astype(_F32)).astype(jnp.int32),
                    axis=0, keepdims=True)
        uidx_ref[0, :, k0:k0 + 448] = jnp.minimum(u, _J - 1) + b * _J


def _s2_kernel(text_ref, sel_ref, wt_ref, wm_ref, m_ref, cnt_ref,
               pf_ref, hard_ref, exp_ref):
    text = text_ref[0]
    sel = sel_ref[0]
    m_row = m_ref[0]
    cnti = cnt_ref[0]
    k_col = jax.lax.broadcasted_iota(jnp.int32, (_K, 1), 0)

    km = _dot_bf(sel, wm_ref[...])
    qt = _dot_bf(text, wt_ref[...])
    energy = _dot_bf_t(qt, km) * (1.0 / 16.0)

    ktot = jnp.max(cnti, axis=1, keepdims=True)
    k_row = jax.lax.broadcasted_iota(jnp.int32, (1, _K), 1)
    valid_row = (k_row.astype(_F32) < ktot)
    em = jnp.where(valid_row, energy, _NEG)
    vmask = valid_row.astype(_F32)

    ex = jnp.exp(em - jnp.max(em, axis=1, keepdims=True))
    mat_p_d = ex / jnp.sum(ex, axis=1, keepdims=True) * vmask

    row_max = jnp.max(em, axis=1, keepdims=True)
    am = jnp.min(jnp.where(em == row_max, k_row, _K), axis=1, keepdims=True)
    hard_ref[0] = jnp.where((k_row == am) & valid_row, 1.0, 0.0)

    ex2 = jnp.exp(em - jnp.max(em, axis=0, keepdims=True))
    w_exp = ex2 / jnp.sum(ex2, axis=0, keepdims=True) * vmask
    exp_ref[0] = jax.lax.dot_general(
        _bf(w_exp), _bf(text), (((0,), (0,)), ((), ())),
        preferred_element_type=_F32)

    cnt_excl = (cnti - m_row).astype(jnp.int32)
    pd_bf = _bf(mat_p_d)
    for c0 in range(0, _J, 256):
        map_bf = jnp.where(
            cnt_excl[:, c0:c0 + 256] == k_col, 1.0, 0.0).astype(_BF)
        pf_ref[0, :, c0:c0 + 256] = jax.lax.dot_general(
            pd_bf, map_bf, (((1,), (0,)), ((), ())),
            preferred_element_type=_F32)


def _stage1(text, mel, wq, wk, wv, wdur):
    out = pl.pallas_call(
        _s1_kernel,
        grid=(_B,),
        in_specs=[
            pl.BlockSpec((1, _I, _CT), lambda b: (b, 0, 0)),
            pl.BlockSpec((1, _J, _CM), lambda b: (b, 0, 0)),
            pl.BlockSpec((_CT, _A), lambda b: (0, 0)),
            pl.BlockSpec((_CM, _A), lambda b: (0, 0)),
            pl.BlockSpec((_CM, _A), lambda b: (0, 0)),
            pl.BlockSpec((1, _A), lambda b: (0, 0)),
        ],
        out_specs=[
            pl.BlockSpec((1, _J, 1), lambda b: (b, 0, 0)),
            pl.BlockSpec((1, _J, 1), lambda b: (b, 0, 0)),
            pl.BlockSpec((1, 1, _K), lambda b: (b, 0, 0)),
        ],
        out_shape=[
            jax.ShapeDtypeStruct((_B, _J, 1), _F32),
            jax.ShapeDtypeStruct((_B, _J, 1), _F32),
            jax.ShapeDtypeStruct((_B, 1, _K), jnp.int32),
        ],
    )(text, mel, wq, wk, wv, wdur.reshape(1, _A))
    return out


def _sc_gather(table, idx):
    nrows = idx.shape[0]
    nw = 32
    per_w = nrows // nw
    ch = 112
    nch = per_w // ch
    mesh = plsc.VectorSubcoreMesh(core_axis_name="c", subcore_axis_name="s")

    @functools.partial(
        pl.kernel, mesh=mesh,
        out_type=jax.ShapeDtypeStruct((nrows, table.shape[1]), _F32),
        scratch_types=[
            pltpu.VMEM((ch,), jnp.int32),
            pltpu.VMEM((ch, table.shape[1]), _F32),
            pltpu.SemaphoreType.DMA,
        ],
    )
    def gk(table_hbm, idx_hbm, out_hbm, idx_v, rows_v, sem):
        wid = jax.lax.axis_index("s") * 2 + jax.lax.axis_index("c")
        base = wid * per_w
        for c in range(nch):
            off = base + c * ch
            pltpu.sync_copy(idx_hbm.at[pl.ds(off, ch)], idx_v)
            pltpu.async_copy(table_hbm.at[idx_v], rows_v, sem).wait()
            pltpu.sync_copy(rows_v, out_hbm.at[pl.ds(off, ch)])

    return gk(table, idx)


def _stage2(text, sel, wt, wm, m2d, cnt2d):
    return pl.pallas_call(
        _s2_kernel,
        grid=(_B,),
        in_specs=[
            pl.BlockSpec((1, _I, _CT), lambda b: (b, 0, 0)),
            pl.BlockSpec((1, _K, _CM), lambda b: (b, 0, 0)),
            pl.BlockSpec((_CT, _A), lambda b: (0, 0)),
            pl.BlockSpec((_CM, _A), lambda b: (0, 0)),
            pl.BlockSpec((1, 1, _J), lambda b: (b, 0, 0)),
            pl.BlockSpec((1, 1, _J), lambda b: (b, 0, 0)),
        ],
        out_specs=[
            pl.BlockSpec((1, _I, _J), lambda b: (b, 0, 0)),
            pl.BlockSpec((1, _I, _K), lambda b: (b, 0, 0)),
            pl.BlockSpec((1, _K, _CT), lambda b: (b, 0, 0)),
        ],
        out_shape=[
            jax.ShapeDtypeStruct((_B, _I, _J), _F32),
            jax.ShapeDtypeStruct((_B, _I, _K), _F32),
            jax.ShapeDtypeStruct((_B, _K, _CT), _F32),
        ],
    )(text, sel, wt, wm, m2d, cnt2d)


def kernel(text_embeddings, mel_embeddings, text_mask, mel_mask,
           Wq, Wk, Wv, w_dur, Wt, Wm):
    m3d, cnt3d, uidx3d = _stage1(
        text_embeddings, mel_embeddings, Wq, Wk, Wv, w_dur)
    m2d = m3d.reshape(_B, 1, _J)
    cnt2d = cnt3d.reshape(_B, 1, _J)
    sel = _sc_gather(
        mel_embeddings.reshape(_B * _J, _CM), uidx3d.reshape(_B * _K)
    ).reshape(_B, _K, _CM)
    mat_p_f, hard, expanded_text = _stage2(
        text_embeddings, sel, Wt, Wm, m2d, cnt2d)
    return mat_p_f, hard, expanded_text
